# baseline (speedup 1.0000x reference)
import jax
import jax.numpy as jnp
from jax import lax
from jax.experimental import pallas as pl
import jax.experimental.pallas.tpu as pltpu
from jax.experimental.pallas import tpu_sc as plsc

VOCAB = 151649
OUTPUT_DIM = 512
MAX_LEN = 300
B = 256

N = B * MAX_LEN
NC, NS = 2, 16
NW = NC * NS
SPAN = N // NW
CHUNK = 40
NCHUNK = SPAN // CHUNK
NGROUP = NCHUNK // 3


def _vgather(v, idx):
    dnums = lax.GatherDimensionNumbers(
        offset_dims=(), collapsed_slice_dims=(0,), start_index_map=(0,))
    return lax.gather(v, idx[:, None], dnums, slice_sizes=(1,),
                      mode=lax.GatherScatterMode.PROMISE_IN_BOUNDS)


def _body(ids_hbm, content_hbm, struct_hbm, out_hbm, out2_hbm, ids_v, c_v,
          struct_v, b0, b1, b2, sg0, sg1, sg2, so0, so1, so2):
    wid = lax.axis_index("s") * NC + lax.axis_index("c")
    n0 = wid * SPAN
    bufs = ((b0, sg0, so0), (b1, sg1, so1), (b2, sg2, so2))

    def start_gather(ch, buf, sem):
        pltpu.async_copy(
            content_hbm.at[ids_v.at[pl.ds(ch * CHUNK, CHUNK)]], buf, sem)

    def wait_gather(ch, buf, sem):
        pltpu.make_async_copy(
            content_hbm.at[ids_v.at[pl.ds(ch * CHUNK, CHUNK)]], buf,
            sem).wait()

    def start_out(ch, buf, sem):
        pltpu.async_copy(buf, out_hbm.at[pl.ds(n0 + ch * CHUNK, CHUNK)], sem)
        pltpu.async_copy(buf, out2_hbm.at[pl.ds(n0 + ch * CHUNK, CHUNK)], sem)

    def wait_out(ch, buf, sem):
        pltpu.make_async_copy(
            buf, out_hbm.at[pl.ds(n0 + ch * CHUNK, CHUNK)], sem).wait()
        pltpu.make_async_copy(
            buf, out2_hbm.at[pl.ds(n0 + ch * CHUNK, CHUNK)], sem).wait()

    pltpu.sync_copy(struct_hbm.at[pl.ds(0, 8)], struct_v)
    pltpu.sync_copy(ids_hbm.at[pl.ds(n0, SPAN)], ids_v)
    start_gather(0, b0, sg0)
    start_gather(1, b1, sg1)
    lane = lax.iota(jnp.int32, 16)
    last = jnp.full((16,), 15, jnp.int32)

    def scan_step(i, c):
        run_c, rs = c
        tok = ids_v[pl.ds(i * 16, 16)]
        pos = lane + i * 16 + n0
        packed = jnp.where(tok > 151645, pos * 4 + (tok - 151645), -1)
        run = packed
        for sh in (1, 2, 4, 8):
            run = jnp.maximum(run, _vgather(run, jnp.maximum(lane - sh, 0)))
        run = jnp.maximum(run, run_c)
        row_start = jnp.where(pos - rs >= MAX_LEN, rs + MAX_LEN, rs)
        c_v[pl.ds(i * 16, 16)] = jnp.where(
            (run >> 2) >= row_start, jnp.bitwise_and(run, 3), 0)
        return _vgather(run, last), _vgather(row_start, last)

    rs0 = jnp.full((16,), 0, jnp.int32) + n0
    lax.fori_loop(0, SPAN // 16, scan_step,
                  (jnp.full((16,), -1, jnp.int32), rs0))

    def group_step(g, _):
        for k, (bk, sgk, sok) in enumerate(bufs):
            ch = g * 3 + k
            t0 = ch * CHUNK
            wait_gather(ch, bk, sgk)

            @plsc.parallel_loop(0, CHUNK)
            def _add(t):
                ct = c_v[pl.ds(t0 + t, 16)][0]
                for j in range(OUTPUT_DIM // 16):
                    d = pl.ds(j * 16, 16)
                    bk[t, d] = bk[t, d] + struct_v[ct, d]

            start_out(ch, bk, sok)
            r = (k + 2) % 3
            br, sgr, sor = bufs[r]
            if k == 0:
                @pl.when(g > 0)
                def _():
                    wait_out(ch - 1, br, sor)
                start_gather(ch + 2, br, sgr)
            else:
                @pl.when(g < NGROUP - 1)
                def _():
                    wait_out(ch - 1, br, sor)
                    start_gather(ch + 2, br, sgr)
        return 0

    lax.fori_loop(0, NGROUP, group_step, 0)
    for k, (bk, sgk, sok) in enumerate(bufs):
        wait_out(NCHUNK - 3 + k, bk, sok)


@jax.jit
def _run(ids_flat, content_table, structure_table):
    mesh = plsc.VectorSubcoreMesh(core_axis_name="c", subcore_axis_name="s")
    return pl.kernel(
        _body,
        out_type=(jax.ShapeDtypeStruct((N, OUTPUT_DIM), jnp.float32),
                  jax.ShapeDtypeStruct((N, OUTPUT_DIM), jnp.float32)),
        mesh=mesh,
        scratch_types=[
            pltpu.VMEM((SPAN,), jnp.int32),
            pltpu.VMEM((SPAN + 16,), jnp.int32),
            pltpu.VMEM((8, OUTPUT_DIM), jnp.float32),
            pltpu.VMEM((CHUNK, OUTPUT_DIM), jnp.float32),
            pltpu.VMEM((CHUNK, OUTPUT_DIM), jnp.float32),
            pltpu.VMEM((CHUNK, OUTPUT_DIM), jnp.float32),
            pltpu.SemaphoreType.DMA,
            pltpu.SemaphoreType.DMA,
            pltpu.SemaphoreType.DMA,
            pltpu.SemaphoreType.DMA,
            pltpu.SemaphoreType.DMA,
            pltpu.SemaphoreType.DMA,
        ],
    )(ids_flat, content_table, structure_table)


def kernel(input_ids, attention_mask, content_table, structure_table):
    ids_flat = input_ids.reshape(N)
    out, out2 = _run(ids_flat, content_table, structure_table)
    return (out.reshape(B, MAX_LEN, OUTPUT_DIM),
            out2.reshape(B, MAX_LEN, OUTPUT_DIM), attention_mask)


# speedup vs baseline: 1.3480x; 1.3480x over previous
import jax
import jax.numpy as jnp
from jax import lax
from jax.experimental import pallas as pl
import jax.experimental.pallas.tpu as pltpu
from jax.experimental.pallas import tpu_sc as plsc

VOCAB = 151649
OUTPUT_DIM = 512
MAX_LEN = 300
B = 256

N = B * MAX_LEN
NC, NS = 2, 16
NW = NC * NS
SPAN = N // NW
CHUNK = 40
NCHUNK = SPAN // CHUNK
NGROUP = NCHUNK // 3


def _vgather(v, idx):
    dnums = lax.GatherDimensionNumbers(
        offset_dims=(), collapsed_slice_dims=(0,), start_index_map=(0,))
    return lax.gather(v, idx[:, None], dnums, slice_sizes=(1,),
                      mode=lax.GatherScatterMode.PROMISE_IN_BOUNDS)


def _body(ids_hbm, content_hbm, struct_hbm, out_hbm, ids_v, c_v,
          struct_v, b0, b1, b2, sg0, sg1, sg2, so0, so1, so2):
    wid = lax.axis_index("s") * NC + lax.axis_index("c")
    n0 = wid * SPAN
    bufs = ((b0, sg0, so0), (b1, sg1, so1), (b2, sg2, so2))

    def start_gather(ch, buf, sem):
        pltpu.async_copy(
            content_hbm.at[ids_v.at[pl.ds(ch * CHUNK, CHUNK)]], buf, sem)

    def wait_gather(ch, buf, sem):
        pltpu.make_async_copy(
            content_hbm.at[ids_v.at[pl.ds(ch * CHUNK, CHUNK)]], buf,
            sem).wait()

    def start_out(ch, buf, sem):
        pltpu.async_copy(buf, out_hbm.at[pl.ds(n0 + ch * CHUNK, CHUNK)], sem)

    def wait_out(ch, buf, sem):
        pltpu.make_async_copy(
            buf, out_hbm.at[pl.ds(n0 + ch * CHUNK, CHUNK)], sem).wait()

    pltpu.sync_copy(struct_hbm.at[pl.ds(0, 8)], struct_v)
    pltpu.sync_copy(ids_hbm.at[pl.ds(n0, SPAN)], ids_v)
    start_gather(0, b0, sg0)
    start_gather(1, b1, sg1)
    lane = lax.iota(jnp.int32, 16)
    last = jnp.full((16,), 15, jnp.int32)

    def scan_step(i, c):
        run_c, rs = c
        tok = ids_v[pl.ds(i * 16, 16)]
        pos = lane + i * 16 + n0
        packed = jnp.where(tok > 151645, pos * 4 + (tok - 151645), -1)
        run = packed
        for sh in (1, 2, 4, 8):
            run = jnp.maximum(run, _vgather(run, jnp.maximum(lane - sh, 0)))
        run = jnp.maximum(run, run_c)
        row_start = jnp.where(pos - rs >= MAX_LEN, rs + MAX_LEN, rs)
        c_v[pl.ds(i * 16, 16)] = jnp.where(
            (run >> 2) >= row_start, jnp.bitwise_and(run, 3), 0)
        return _vgather(run, last), _vgather(row_start, last)

    rs0 = jnp.full((16,), 0, jnp.int32) + n0
    lax.fori_loop(0, SPAN // 16, scan_step,
                  (jnp.full((16,), -1, jnp.int32), rs0))

    def group_step(g, _):
        for k, (bk, sgk, sok) in enumerate(bufs):
            ch = g * 3 + k
            t0 = ch * CHUNK
            wait_gather(ch, bk, sgk)

            @plsc.parallel_loop(0, CHUNK)
            def _add(t):
                ct = c_v[pl.ds(t0 + t, 16)][0]
                for j in range(OUTPUT_DIM // 16):
                    d = pl.ds(j * 16, 16)
                    bk[t, d] = bk[t, d] + struct_v[ct, d]

            start_out(ch, bk, sok)
            r = (k + 2) % 3
            br, sgr, sor = bufs[r]
            if k == 0:
                @pl.when(g > 0)
                def _():
                    wait_out(ch - 1, br, sor)
                start_gather(ch + 2, br, sgr)
            else:
                @pl.when(g < NGROUP - 1)
                def _():
                    wait_out(ch - 1, br, sor)
                    start_gather(ch + 2, br, sgr)
        return 0

    lax.fori_loop(0, NGROUP, group_step, 0)
    for k, (bk, sgk, sok) in enumerate(bufs):
        wait_out(NCHUNK - 3 + k, bk, sok)


@jax.jit
def _run(ids_flat, content_table, structure_table):
    mesh = plsc.VectorSubcoreMesh(core_axis_name="c", subcore_axis_name="s")
    return pl.kernel(
        _body,
        out_type=jax.ShapeDtypeStruct((N, OUTPUT_DIM), jnp.float32),
        mesh=mesh,
        scratch_types=[
            pltpu.VMEM((SPAN,), jnp.int32),
            pltpu.VMEM((SPAN + 16,), jnp.int32),
            pltpu.VMEM((8, OUTPUT_DIM), jnp.float32),
            pltpu.VMEM((CHUNK, OUTPUT_DIM), jnp.float32),
            pltpu.VMEM((CHUNK, OUTPUT_DIM), jnp.float32),
            pltpu.VMEM((CHUNK, OUTPUT_DIM), jnp.float32),
            pltpu.SemaphoreType.DMA,
            pltpu.SemaphoreType.DMA,
            pltpu.SemaphoreType.DMA,
            pltpu.SemaphoreType.DMA,
            pltpu.SemaphoreType.DMA,
            pltpu.SemaphoreType.DMA,
        ],
    )(ids_flat, content_table, structure_table)


def kernel(input_ids, attention_mask, content_table, structure_table):
    ids_flat = input_ids.reshape(N)
    out = _run(ids_flat, content_table, structure_table)
    out3 = out.reshape(B, MAX_LEN, OUTPUT_DIM)
    return (out3, out3, attention_mask)
